# baseline (speedup 1.0000x reference)
import functools

import jax
import jax.numpy as jnp
from jax import lax
from jax.experimental import pallas as pl
from jax.experimental.pallas import tpu as pltpu
from jax.experimental.pallas import tpu_sc as plsc

_N = 10000
_E = 320000
_D = 128
_NC = 2
_NS = 16
_NW = _NC * _NS
_C = 125
_NCH = 80
_NPH = 4
_PH = _NCH // _NPH
_RT0 = 624
_RT1 = 640
_BASE1 = _RT0 * (_NS - 1)
_BM = 2000
_ERT0 = 640
_NPAD = _ERT0 * _NS


def _per_tile_slab(sid, copy_fn):
    @pl.when(sid < _NS - 1)
    def _():
        copy_fn(sid * _RT0, _RT0)

    @pl.when(sid == _NS - 1)
    def _():
        copy_fn(_BASE1, _RT1)


def _sc_mesh():
    return plsc.VectorSubcoreMesh(
        core_axis_name="c", subcore_axis_name="s",
        num_cores=_NC, num_subcores=_NS)


def _deg_body(dst_hbm, zeros_hbm, out_hbm, idx_v, ones_v, acc_sh, sem):
    cid = lax.axis_index("c")
    sid = lax.axis_index("s")
    wid = cid * _NS + sid
    base = sid * _ERT0
    for i in range(8):
        ones_v[pl.ds(16 * i, 16)] = jnp.ones((16,), jnp.float32)
    pltpu.sync_copy(zeros_hbm, acc_sh.at[pl.ds(base, _ERT0)])
    pltpu.sync_copy(dst_hbm.at[wid], idx_v)
    plsc.subcore_barrier()

    def chunk(t, carry):
        p = t // _PH
        j = t - p * _PH
        pltpu.async_copy(ones_v.at[pl.ds(0, _C)],
                         acc_sh.at[idx_v.at[p, j]], sem, add=True)
        return carry

    lax.fori_loop(0, _NCH, chunk, 0)

    def drain(t, carry):
        p = t // _PH
        j = t - p * _PH
        pltpu.make_async_copy(ones_v.at[pl.ds(0, _C)],
                              acc_sh.at[idx_v.at[p, j]], sem).wait()
        return carry

    lax.fori_loop(0, _NCH, drain, 0)
    plsc.subcore_barrier()
    pltpu.sync_copy(acc_sh.at[pl.ds(base, _ERT0)],
                    out_hbm.at[cid, pl.ds(base, _ERT0)])


@functools.cache
def _deg_call():
    return pl.kernel(
        _deg_body,
        out_type=jax.ShapeDtypeStruct((_NC, _NPAD), jnp.float32),
        mesh=_sc_mesh(),
        scratch_types=[
            pltpu.VMEM((_NPH, _PH, _C), jnp.int32),
            pltpu.VMEM((128,), jnp.float32),
            pltpu.VMEM_SHARED((_NPAD,), jnp.float32),
            pltpu.SemaphoreType.DMA,
        ],
    )


def _scat_body(g_hbm, src_hbm, dst_hbm, zeros_hbm, out_hbm,
               src_v, dst_v, rows_v, acc_sh, sem0, sem1, isem0, isem1):
    cid = lax.axis_index("c")
    sid = lax.axis_index("s")
    wid = cid * _NS + sid

    @pl.when(cid == 0)
    def _():
        _per_tile_slab(sid, lambda b, n: pltpu.sync_copy(
            g_hbm.at[pl.ds(b, n)], acc_sh.at[pl.ds(b, n)]))

    @pl.when(cid != 0)
    def _():
        _per_tile_slab(sid, lambda b, n: pltpu.sync_copy(
            zeros_hbm.at[pl.ds(0, n)], acc_sh.at[pl.ds(b, n)]))

    plsc.subcore_barrier()

    def idx_load(p, j, sem):
        pltpu.async_copy(src_hbm.at[wid, p], src_v.at[j], sem)
        pltpu.async_copy(dst_hbm.at[wid, p], dst_v.at[j], sem)

    def idx_wait(p, j, sem):
        pltpu.make_async_copy(src_hbm.at[wid, p], src_v.at[j], sem).wait()
        pltpu.make_async_copy(dst_hbm.at[wid, p], dst_v.at[j], sem).wait()

    isems = (isem0, isem1)
    idx_load(0, 0, isem0)

    def outer(i, carry):
        for j in range(2):
            p = 2 * i + j
            idx_wait(p, j, isems[j])

            @pl.when(p + 1 < _NPH)
            def _():
                idx_load(p + 1, 1 - j, isems[1 - j])

            sv = src_v.at[j]
            dv = dst_v.at[j]
            pltpu.async_copy(g_hbm.at[sv.at[0]], rows_v.at[0], sem0)
            pltpu.async_copy(g_hbm.at[sv.at[1]], rows_v.at[1], sem1)

            def body(k, c, sv=sv, dv=dv):
                a = 2 * k
                b = a + 1
                pltpu.make_async_copy(
                    g_hbm.at[sv.at[a]], rows_v.at[0], sem0).wait()

                @pl.when(k > 0)
                def _():
                    pltpu.async_copy(g_hbm.at[sv.at[b]], rows_v.at[1], sem1)

                pltpu.sync_copy(rows_v.at[0], acc_sh.at[dv.at[a]], add=True)
                pltpu.make_async_copy(
                    g_hbm.at[sv.at[b]], rows_v.at[1], sem1).wait()

                @pl.when(k + 1 < _PH // 2)
                def _():
                    pltpu.async_copy(g_hbm.at[sv.at[a + 2]],
                                     rows_v.at[0], sem0)

                pltpu.sync_copy(rows_v.at[1], acc_sh.at[dv.at[b]], add=True)
                return c

            lax.fori_loop(0, _PH // 2, body, 0)
        return carry

    lax.fori_loop(0, _NPH // 2, outer, 0)
    plsc.subcore_barrier()
    _per_tile_slab(sid, lambda b, n: pltpu.sync_copy(
        acc_sh.at[pl.ds(b, n)], out_hbm.at[cid, pl.ds(b, n)]))


@functools.cache
def _scat_call():
    return pl.kernel(
        _scat_body,
        out_type=jax.ShapeDtypeStruct((_NC, _N, _D), jnp.float32),
        mesh=_sc_mesh(),
        scratch_types=[
            pltpu.VMEM((2, _PH, _C), jnp.int32),
            pltpu.VMEM((2, _PH, _C), jnp.int32),
            pltpu.VMEM((2, _C, _D), jnp.float32),
            pltpu.VMEM_SHARED((_N, _D), jnp.float32),
            pltpu.SemaphoreType.DMA,
            pltpu.SemaphoreType.DMA,
            pltpu.SemaphoreType.DMA,
            pltpu.SemaphoreType.DMA,
        ],
    )


def _stage1_body(degp_ref, x_ref, w1_ref, g1_ref, dis_ref):
    degp = degp_ref[...]
    deg = degp[:, 0:1] + degp[:, 1:2] + 1.0
    dis = lax.rsqrt(deg)
    dis_ref[...] = dis
    g1_ref[...] = jnp.dot(x_ref[...], w1_ref[...],
                          preferred_element_type=jnp.float32) * dis


def _stage2_body(y_ref, dis_ref, b1_ref, w2_ref, g2_ref):
    y = y_ref[...]
    dis = dis_ref[...]
    h = jnp.maximum(dis * (y[0] + y[1]) + b1_ref[...], 0.0)
    g2_ref[...] = jnp.dot(h, w2_ref[...],
                          preferred_element_type=jnp.float32) * dis


def _stage3_body(y_ref, dis_ref, b2_ref, wd1_ref, bd1_ref, wd2_ref, bd2_ref,
                 o_ref):
    y = y_ref[...]
    dis = dis_ref[...]
    h2 = jnp.maximum(dis * (y[0] + y[1]) + b2_ref[...], 0.0)
    h3 = jnp.maximum(jnp.dot(h2, wd1_ref[...],
                             preferred_element_type=jnp.float32)
                     + bd1_ref[...], 0.0)
    o_ref[...] = jnp.dot(h3, wd2_ref[...],
                         preferred_element_type=jnp.float32) + bd2_ref[...]


def _full(shape):
    return pl.BlockSpec(shape, lambda i: (0,) * len(shape))


def _rows(shape, dim=0):
    def idx(i):
        out = [0] * len(shape)
        out[dim] = i
        return tuple(out)
    return pl.BlockSpec(shape, idx)


@functools.cache
def _stage1_call():
    return pl.pallas_call(
        _stage1_body,
        grid=(_N // _BM,),
        in_specs=[_rows((_BM, _NC)), _rows((_BM, _D)), _full((_D, _D))],
        out_specs=[_rows((_BM, _D)), _rows((_BM, 1))],
        out_shape=[jax.ShapeDtypeStruct((_N, _D), jnp.float32),
                   jax.ShapeDtypeStruct((_N, 1), jnp.float32)],
    )


@functools.cache
def _stage2_call():
    return pl.pallas_call(
        _stage2_body,
        grid=(_N // _BM,),
        in_specs=[_rows((_NC, _BM, _D), 1), _rows((_BM, 1)),
                  _full((1, _D)), _full((_D, _D))],
        out_specs=_rows((_BM, _D)),
        out_shape=jax.ShapeDtypeStruct((_N, _D), jnp.float32),
    )


@functools.cache
def _stage3_call():
    return pl.pallas_call(
        _stage3_body,
        grid=(_N // _BM,),
        in_specs=[_rows((_NC, _BM, _D), 1), _rows((_BM, 1)),
                  _full((1, _D)), _full((_D, _D)), _full((1, _D)),
                  _full((_D, _D)), _full((1, _D))],
        out_specs=_rows((_BM, _D)),
        out_shape=jax.ShapeDtypeStruct((_N, _D), jnp.float32),
    )


def kernel(x, edge_index, W1, b1, W2, b2, Wd1, bd1, Wd2, bd2):
    src = edge_index[0].reshape(_NW, _NPH, _PH, _C)
    dst = edge_index[1].reshape(_NW, _NPH, _PH, _C)
    zeros_d = jnp.zeros((_RT1, _D), jnp.float32)
    zeros_w = jnp.zeros((_ERT0,), jnp.float32)

    degp = _deg_call()(dst, zeros_w)
    g1, dis = _stage1_call()(degp[:, :_N].T, x, W1)
    y1 = _scat_call()(g1, src, dst, zeros_d)
    g2 = _stage2_call()(y1, dis, b1.reshape(1, _D), W2)
    y2 = _scat_call()(g2, src, dst, zeros_d)
    return _stage3_call()(y2, dis, b2.reshape(1, _D), Wd1,
                          bd1.reshape(1, _D), Wd2, bd2.reshape(1, _D))


# speedup vs baseline: 1.0059x; 1.0059x over previous
import functools

import jax
import jax.numpy as jnp
from jax import lax
from jax.experimental import pallas as pl
from jax.experimental.pallas import tpu as pltpu
from jax.experimental.pallas import tpu_sc as plsc

_N = 10000
_E = 320000
_D = 128
_NC = 2
_NS = 16
_NW = _NC * _NS
_C = 125
_NCH = 80
_NPH = 4
_PH = _NCH // _NPH
_RT0 = 624
_RT1 = 640
_BASE1 = _RT0 * (_NS - 1)
_BM = 2000
_ERT0 = 640
_NPAD = _ERT0 * _NS


def _per_tile_slab(sid, copy_fn):
    @pl.when(sid < _NS - 1)
    def _():
        copy_fn(sid * _RT0, _RT0)

    @pl.when(sid == _NS - 1)
    def _():
        copy_fn(_BASE1, _RT1)


def _sc_mesh():
    return plsc.VectorSubcoreMesh(
        core_axis_name="c", subcore_axis_name="s",
        num_cores=_NC, num_subcores=_NS)


def _deg_body(dst_hbm, zeros_hbm, out_hbm, idx_v, ones_v, acc_sh, sem):
    cid = lax.axis_index("c")
    sid = lax.axis_index("s")
    wid = cid * _NS + sid
    base = sid * _ERT0
    for i in range(8):
        ones_v[pl.ds(16 * i, 16)] = jnp.ones((16,), jnp.float32)
    pltpu.sync_copy(zeros_hbm, acc_sh.at[pl.ds(base, _ERT0)])
    pltpu.sync_copy(dst_hbm.at[wid], idx_v)
    plsc.subcore_barrier()

    def chunk(t, carry):
        p = t // _PH
        j = t - p * _PH
        pltpu.async_copy(ones_v.at[pl.ds(0, _C)],
                         acc_sh.at[idx_v.at[p, j]], sem, add=True)
        return carry

    lax.fori_loop(0, _NCH, chunk, 0)

    def drain(t, carry):
        p = t // _PH
        j = t - p * _PH
        pltpu.make_async_copy(ones_v.at[pl.ds(0, _C)],
                              acc_sh.at[idx_v.at[p, j]], sem).wait()
        return carry

    lax.fori_loop(0, _NCH, drain, 0)
    plsc.subcore_barrier()
    pltpu.sync_copy(acc_sh.at[pl.ds(base, _ERT0)],
                    out_hbm.at[cid, pl.ds(base, _ERT0)])


@functools.cache
def _deg_call():
    return pl.kernel(
        _deg_body,
        out_type=jax.ShapeDtypeStruct((_NC, _NPAD), jnp.float32),
        mesh=_sc_mesh(),
        scratch_types=[
            pltpu.VMEM((_NPH, _PH, _C), jnp.int32),
            pltpu.VMEM((128,), jnp.float32),
            pltpu.VMEM_SHARED((_NPAD,), jnp.float32),
            pltpu.SemaphoreType.DMA,
        ],
    )


def _scat_body(g_hbm, src_hbm, dst_hbm, zeros_hbm, out_hbm,
               src_v, dst_v, rows_v, acc_sh, sem0, sem1, isem0, isem1):
    cid = lax.axis_index("c")
    sid = lax.axis_index("s")
    wid = cid * _NS + sid

    @pl.when(cid == 0)
    def _():
        _per_tile_slab(sid, lambda b, n: pltpu.sync_copy(
            g_hbm.at[pl.ds(b, n)], acc_sh.at[pl.ds(b, n)]))

    @pl.when(cid != 0)
    def _():
        _per_tile_slab(sid, lambda b, n: pltpu.sync_copy(
            zeros_hbm.at[pl.ds(0, n)], acc_sh.at[pl.ds(b, n)]))

    plsc.subcore_barrier()

    def idx_load(p, j, sem):
        pltpu.async_copy(src_hbm.at[wid, p], src_v.at[j], sem)
        pltpu.async_copy(dst_hbm.at[wid, p], dst_v.at[j], sem)

    def idx_wait(p, j, sem):
        pltpu.make_async_copy(src_hbm.at[wid, p], src_v.at[j], sem).wait()
        pltpu.make_async_copy(dst_hbm.at[wid, p], dst_v.at[j], sem).wait()

    isems = (isem0, isem1)
    idx_load(0, 0, isem0)

    def outer(i, carry):
        for j in range(2):
            p = 2 * i + j
            idx_wait(p, j, isems[j])

            @pl.when(p + 1 < _NPH)
            def _():
                idx_load(p + 1, 1 - j, isems[1 - j])

            sv = src_v.at[j]
            dv = dst_v.at[j]
            pltpu.async_copy(g_hbm.at[sv.at[0]], rows_v.at[0], sem0)

            def body(k, c, sv=sv, dv=dv):
                a = 2 * k
                b = a + 1
                pltpu.make_async_copy(
                    g_hbm.at[sv.at[a]], rows_v.at[0], sem0).wait()
                pltpu.async_copy(g_hbm.at[sv.at[b]], rows_v.at[1], sem1)
                pltpu.sync_copy(rows_v.at[0], acc_sh.at[dv.at[a]], add=True)
                pltpu.make_async_copy(
                    g_hbm.at[sv.at[b]], rows_v.at[1], sem1).wait()

                @pl.when(k + 1 < _PH // 2)
                def _():
                    pltpu.async_copy(g_hbm.at[sv.at[a + 2]],
                                     rows_v.at[0], sem0)

                pltpu.sync_copy(rows_v.at[1], acc_sh.at[dv.at[b]], add=True)
                return c

            lax.fori_loop(0, _PH // 2, body, 0)
        return carry

    lax.fori_loop(0, _NPH // 2, outer, 0)
    plsc.subcore_barrier()
    _per_tile_slab(sid, lambda b, n: pltpu.sync_copy(
        acc_sh.at[pl.ds(b, n)], out_hbm.at[cid, pl.ds(b, n)]))


@functools.cache
def _scat_call():
    return pl.kernel(
        _scat_body,
        out_type=jax.ShapeDtypeStruct((_NC, _N, _D), jnp.float32),
        mesh=_sc_mesh(),
        scratch_types=[
            pltpu.VMEM((2, _PH, _C), jnp.int32),
            pltpu.VMEM((2, _PH, _C), jnp.int32),
            pltpu.VMEM((2, _C, _D), jnp.float32),
            pltpu.VMEM_SHARED((_N, _D), jnp.float32),
            pltpu.SemaphoreType.DMA,
            pltpu.SemaphoreType.DMA,
            pltpu.SemaphoreType.DMA,
            pltpu.SemaphoreType.DMA,
        ],
    )


def _stage1_body(degp_ref, x_ref, w1_ref, g1_ref, dis_ref):
    degp = degp_ref[...]
    deg = degp[:, 0:1] + degp[:, 1:2] + 1.0
    dis = lax.rsqrt(deg)
    dis_ref[...] = dis
    g1_ref[...] = jnp.dot(x_ref[...], w1_ref[...],
                          preferred_element_type=jnp.float32) * dis


def _stage2_body(y_ref, dis_ref, b1_ref, w2_ref, g2_ref):
    y = y_ref[...]
    dis = dis_ref[...]
    h = jnp.maximum(dis * (y[0] + y[1]) + b1_ref[...], 0.0)
    g2_ref[...] = jnp.dot(h, w2_ref[...],
                          preferred_element_type=jnp.float32) * dis


def _stage3_body(y_ref, dis_ref, b2_ref, wd1_ref, bd1_ref, wd2_ref, bd2_ref,
                 o_ref):
    y = y_ref[...]
    dis = dis_ref[...]
    h2 = jnp.maximum(dis * (y[0] + y[1]) + b2_ref[...], 0.0)
    h3 = jnp.maximum(jnp.dot(h2, wd1_ref[...],
                             preferred_element_type=jnp.float32)
                     + bd1_ref[...], 0.0)
    o_ref[...] = jnp.dot(h3, wd2_ref[...],
                         preferred_element_type=jnp.float32) + bd2_ref[...]


def _full(shape):
    return pl.BlockSpec(shape, lambda i: (0,) * len(shape))


def _rows(shape, dim=0):
    def idx(i):
        out = [0] * len(shape)
        out[dim] = i
        return tuple(out)
    return pl.BlockSpec(shape, idx)


@functools.cache
def _stage1_call():
    return pl.pallas_call(
        _stage1_body,
        grid=(_N // _BM,),
        in_specs=[_rows((_BM, _NC)), _rows((_BM, _D)), _full((_D, _D))],
        out_specs=[_rows((_BM, _D)), _rows((_BM, 1))],
        out_shape=[jax.ShapeDtypeStruct((_N, _D), jnp.float32),
                   jax.ShapeDtypeStruct((_N, 1), jnp.float32)],
    )


@functools.cache
def _stage2_call():
    return pl.pallas_call(
        _stage2_body,
        grid=(_N // _BM,),
        in_specs=[_rows((_NC, _BM, _D), 1), _rows((_BM, 1)),
                  _full((1, _D)), _full((_D, _D))],
        out_specs=_rows((_BM, _D)),
        out_shape=jax.ShapeDtypeStruct((_N, _D), jnp.float32),
    )


@functools.cache
def _stage3_call():
    return pl.pallas_call(
        _stage3_body,
        grid=(_N // _BM,),
        in_specs=[_rows((_NC, _BM, _D), 1), _rows((_BM, 1)),
                  _full((1, _D)), _full((_D, _D)), _full((1, _D)),
                  _full((_D, _D)), _full((1, _D))],
        out_specs=_rows((_BM, _D)),
        out_shape=jax.ShapeDtypeStruct((_N, _D), jnp.float32),
    )


def kernel(x, edge_index, W1, b1, W2, b2, Wd1, bd1, Wd2, bd2):
    src = edge_index[0].reshape(_NW, _NPH, _PH, _C)
    dst = edge_index[1].reshape(_NW, _NPH, _PH, _C)
    zeros_d = jnp.zeros((_RT1, _D), jnp.float32)
    zeros_w = jnp.zeros((_ERT0,), jnp.float32)

    degp = _deg_call()(dst, zeros_w)
    g1, dis = _stage1_call()(degp[:, :_N].T, x, W1)
    y1 = _scat_call()(g1, src, dst, zeros_d)
    g2 = _stage2_call()(y1, dis, b1.reshape(1, _D), W2)
    y2 = _scat_call()(g2, src, dst, zeros_d)
    return _stage3_call()(y2, dis, b2.reshape(1, _D), Wd1,
                          bd1.reshape(1, _D), Wd2, bd2.reshape(1, _D))
